# Initial kernel scaffold; baseline (speedup 1.0000x reference)
import jax
import jax.numpy as jnp
from jax.experimental import pallas as pl


def kernel(user_feats, graph_node_features, graph_edge_index, merged_tree_feature, merged_tree_edge_index, indices, tweet_embedding, ue_w1, ue_b1, ue_w2, ue_b2, wih0, whh0, bih0, bhh0, wih1, whh1, bih1, bhh1, conv1_w, conv1_b, convm_w, convm_b, convv_w, convv_b, fc1_w, fc1_b):
    raise NotImplementedError("write your pallas kernel here")


# speedup vs baseline: 1.9600x; 1.9600x over previous
import functools
import jax
import jax.numpy as jnp
from jax import lax
from jax.experimental import pallas as pl
from jax.experimental.pallas import tpu as pltpu
from jax.experimental.pallas import tpu_sc as plsc

N = 4096
NT = 3072
NU = 1024
E = 65536
SEQ = 24
VOCAB = 30000
B = 64
DP = 128
HIGH = lax.Precision.HIGHEST


_NW = 32
_ROWS = NT * SEQ
_RPW = _ROWS // _NW
_CHUNKS = _RPW // 128


def _sc_gather(table, ids2d):
  mesh = plsc.VectorSubcoreMesh(core_axis_name="c", subcore_axis_name="s")

  @functools.partial(
      pl.kernel,
      mesh=mesh,
      out_type=jax.ShapeDtypeStruct((_ROWS, DP), jnp.float32),
      scratch_types=[
          pltpu.VMEM((_CHUNKS, 128), jnp.int32),
          pltpu.VMEM((128, DP), jnp.float32),
          pltpu.SemaphoreType.DMA,
      ],
  )
  def k(table_hbm, idx_hbm, out_hbm, idx_v, rows_v, sem):
    wid = lax.axis_index("s") * 2 + lax.axis_index("c")
    pltpu.sync_copy(idx_hbm.at[wid], idx_v)
    for c in range(_CHUNKS):
      pltpu.async_copy(table_hbm.at[idx_v.at[c]], rows_v, sem).wait()
      pltpu.sync_copy(
          rows_v, out_hbm.at[pl.ds(wid * _RPW + c * 128, 128)])

  return k(table, ids2d)


_BLK_ELEMS = 256 * N
_EPT = E // 16
_ZB = 16384


def _sc_build_c(edges_flat):
  mesh = plsc.VectorSubcoreMesh(core_axis_name="c", subcore_axis_name="s")

  @functools.partial(
      pl.kernel,
      mesh=mesh,
      out_type=jax.ShapeDtypeStruct((N * N,), jnp.float32),
      scratch_types=[
          pltpu.VMEM((_EPT,), jnp.int32),
          pltpu.VMEM((_EPT,), jnp.int32),
          pltpu.VMEM((_EPT,), jnp.int32),
          pltpu.VMEM((32, 128), jnp.int32),
          pltpu.VMEM((_EPT,), jnp.float32),
          pltpu.VMEM((_ZB,), jnp.float32),
          pltpu.VMEM_SHARED((_BLK_ELEMS + 16,), jnp.float32),
      ],
  )
  def k(edges_hbm, out_hbm, srcv, dstv, keyv, idxv, onesv, zerov, acc):
    cid = lax.axis_index("c")
    sid = lax.axis_index("s")
    ebase = sid * _EPT
    pltpu.sync_copy(edges_hbm.at[pl.ds(ebase, _EPT)], srcv)
    pltpu.sync_copy(edges_hbm.at[pl.ds(E + ebase, _EPT)], dstv)

    def init_body(i, _):
      sl = pl.ds(i * 16, 16)
      keyv[sl] = srcv[sl] * N + dstv[sl]
      onesv[sl] = jnp.full((16,), 1.0, jnp.float32)
      return 0

    lax.fori_loop(0, _EPT // 16, init_body, 0)

    def zero_body(i, _):
      zerov[pl.ds(i * 16, 16)] = jnp.zeros((16,), jnp.float32)
      return 0

    lax.fori_loop(0, _ZB // 16, zero_body, 0)

    for p in range(8):
      blk = p * 2 + cid
      base = blk * _BLK_ELEMS
      for q in range(_BLK_ELEMS // 16 // _ZB):
        pltpu.sync_copy(
            zerov,
            acc.at[pl.ds(sid * (_BLK_ELEMS // 16) + q * _ZB, _ZB)])
      @pl.when(sid == 0)
      def _():
        pltpu.sync_copy(zerov.at[pl.ds(0, 16)], acc.at[pl.ds(_BLK_ELEMS, 16)])
      plsc.subcore_barrier()

      def mask_body(r, _):
        for q in range(8):
          sl = pl.ds(r * 128 + q * 16, 16)
          kk = keyv[sl] - base
          ok = (kk >= 0) & (kk < _BLK_ELEMS)
          idxv[r, pl.ds(q * 16, 16)] = jnp.where(
              ok, kk, jnp.full((16,), _BLK_ELEMS, jnp.int32))
        return 0

      lax.fori_loop(0, 32, mask_body, 0)

      for j in range(32):
        pltpu.sync_copy(
            onesv.at[pl.ds(j * 128, 128)], acc.at[idxv.at[j]], add=True)
      plsc.subcore_barrier()
      pltpu.sync_copy(
          acc.at[pl.ds(sid * (_BLK_ELEMS // 16), _BLK_ELEMS // 16)],
          out_hbm.at[pl.ds(base + sid * (_BLK_ELEMS // 16),
                           _BLK_ELEMS // 16)])
      plsc.subcore_barrier()

  return k(edges_flat)


def _user_mlp_kernel(uf_ref, w1_ref, b1_ref, w2_ref, b2_ref, out_ref):
  u = jnp.maximum(
      jnp.dot(uf_ref[...], w1_ref[...], precision=HIGH) + b1_ref[...], 0.0)
  out_ref[...] = jnp.dot(u, w2_ref[...], precision=HIGH) + b2_ref[...]


def _user_mlp(uf, w1t, b1, w2t, b2):
  return pl.pallas_call(
      _user_mlp_kernel,
      out_shape=jax.ShapeDtypeStruct((NU, DP), jnp.float32),
  )(uf, w1t, b1, w2t, b2)


_GB = 512


def _gru_kernel(emb_ref, h00_ref, h01_ref,
                w0r, w0z, w0n, u0r, u0z, u0n, b0r, b0z, b0i, b0h,
                w1r, w1z, w1n, u1r, u1z, u1n, b1r, b1z, b1i, b1h,
                hn_ref, out0_s):
  x = emb_ref[...].reshape(_GB, SEQ, DP)
  h = h00_ref[...]
  for t in range(SEQ):
    xt = x[:, t, :]
    r = jax.nn.sigmoid(jnp.dot(xt, w0r[...], precision=HIGH) +
                       jnp.dot(h, u0r[...], precision=HIGH) + b0r[...])
    z = jax.nn.sigmoid(jnp.dot(xt, w0z[...], precision=HIGH) +
                       jnp.dot(h, u0z[...], precision=HIGH) + b0z[...])
    n = jnp.tanh(jnp.dot(xt, w0n[...], precision=HIGH) + b0i[...] +
                 r * (jnp.dot(h, u0n[...], precision=HIGH) + b0h[...]))
    h = (1.0 - z) * n + z * h
    out0_s[:, t, :] = h
  h = h01_ref[...]
  for t in range(SEQ):
    ot = out0_s[:, t, :]
    r = jax.nn.sigmoid(jnp.dot(ot, w1r[...], precision=HIGH) +
                       jnp.dot(h, u1r[...], precision=HIGH) + b1r[...])
    z = jax.nn.sigmoid(jnp.dot(ot, w1z[...], precision=HIGH) +
                       jnp.dot(h, u1z[...], precision=HIGH) + b1z[...])
    n = jnp.tanh(jnp.dot(ot, w1n[...], precision=HIGH) + b1i[...] +
                 r * (jnp.dot(h, u1n[...], precision=HIGH) + b1h[...]))
    h = (1.0 - z) * n + z * h
  hn_ref[...] = h


def _gru(emb, h00, h01, wmats, bvecs):
  wspec = pl.BlockSpec((DP, DP), lambda j: (0, 0))
  bspec = pl.BlockSpec((1, DP), lambda j: (0, 0))
  return pl.pallas_call(
      _gru_kernel,
      grid=(NT // _GB,),
      in_specs=[
          pl.BlockSpec((_GB * SEQ, DP), lambda j: (j, 0)),
          pl.BlockSpec((_GB, DP), lambda j: (j, 0)),
          pl.BlockSpec((_GB, DP), lambda j: (j, 0)),
      ] + [wspec] * 6 + [bspec] * 4 + [wspec] * 6 + [bspec] * 4,
      out_specs=pl.BlockSpec((_GB, DP), lambda j: (j, 0)),
      out_shape=jax.ShapeDtypeStruct((NT, DP), jnp.float32),
      scratch_shapes=[pltpu.VMEM((_GB, SEQ, DP), jnp.float32)],
  )(emb, h00, h01, *wmats[:6], *bvecs[:4], *wmats[6:], *bvecs[4:])


_RB = 512


def _deg_kernel(c_ref, dinv_ref, suma_ref):
  j = pl.program_id(0)

  @pl.when(j == 0)
  def _():
    dinv_ref[...] = jnp.zeros_like(dinv_ref)

  dinv_ref[...] += jnp.sum(c_ref[...], axis=0, keepdims=True)

  @pl.when(j == N // _RB - 1)
  def _():
    cs = dinv_ref[...]
    suma_ref[...] = (jnp.sum(cs) + float(N)).reshape(1, 1)
    dinv_ref[...] = lax.rsqrt(cs + 1.0)


def _deg(c):
  return pl.pallas_call(
      _deg_kernel,
      grid=(N // _RB,),
      in_specs=[pl.BlockSpec((_RB, N), lambda j: (j, 0))],
      out_specs=[
          pl.BlockSpec((1, N), lambda j: (0, 0)),
          pl.BlockSpec((1, 1), lambda j: (0, 0)),
      ],
      out_shape=[
          jax.ShapeDtypeStruct((1, N), jnp.float32),
          jax.ShapeDtypeStruct((1, 1), jnp.float32),
      ],
  )(c)


def _make_gcn(act):
  def body(c_ref, x_ref, w_ref, b_ref, dall_ref, dblk_ref, out_ref, hd_s):
    j = pl.program_id(0)

    @pl.when(j == 0)
    def _():
      hd_s[...] = jnp.dot(x_ref[...], w_ref[...],
                          precision=HIGH) * dall_ref[...]

    y = lax.dot_general(c_ref[...], hd_s[...], (((0,), (0,)), ((), ())),
                        precision=HIGH)
    y = (y + hd_s[pl.ds(j * _RB, _RB), :]) * dblk_ref[...] + b_ref[...]
    out_ref[...] = act(y)

  def run(c, x, w, b, dinv_col):
    return pl.pallas_call(
        body,
        grid=(N // _RB,),
        in_specs=[
            pl.BlockSpec((N, _RB), lambda j: (0, j)),
            pl.BlockSpec((N, DP), lambda j: (0, 0)),
            pl.BlockSpec((DP, DP), lambda j: (0, 0)),
            pl.BlockSpec((1, DP), lambda j: (0, 0)),
            pl.BlockSpec((N, 1), lambda j: (0, 0)),
            pl.BlockSpec((_RB, 1), lambda j: (j, 0)),
        ],
        out_specs=pl.BlockSpec((_RB, DP), lambda j: (j, 0)),
        out_shape=jax.ShapeDtypeStruct((N, DP), jnp.float32),
        scratch_shapes=[pltpu.VMEM((N, DP), jnp.float32)],
    )(c, x, w, b, dinv_col, dinv_col)

  return run


_gcn_elu = _make_gcn(lambda y: jnp.where(y > 0, y, jnp.exp(y) - 1.0))
_gcn_relu = _make_gcn(lambda y: jnp.maximum(y, 0.0))


def _final_kernel(mu_ref, lv_ref, eps_ref, c_ref, oh_ref, fw_ref, fb_ref,
                  pw_ref, oy_ref, kl_ref, rec_ref, z_s, acc):
  i = pl.program_id(0)
  nblk = N // _RB

  @pl.when(i == 0)
  def _():
    mu = mu_ref[...]
    lv = lv_ref[...]
    z_s[...] = mu + jnp.exp(lv * 0.5) * eps_ref[...]
    klds = -0.5 * (1.0 + lv - mu * mu - jnp.exp(lv))
    acc[0] = 0.0
    acc[1] = 0.0
    acc[2] = jnp.sum(klds)

  zb = z_s[pl.ds(i * _RB, _RB), :]
  logits = lax.dot_general(zb, z_s[...], (((1,), (1,)), ((), ())),
                           precision=HIGH)
  p = jax.nn.sigmoid(logits)
  acc[0] += jnp.sum(p)
  cols = lax.broadcasted_iota(jnp.int32, (_RB, N), 1)
  rows = lax.broadcasted_iota(jnp.int32, (_RB, N), 0) + i * _RB
  t = c_ref[...] + jnp.where(cols == rows, 1.0, 0.0)
  logp = jnp.maximum(jnp.log(p), -100.0)
  log1p = jnp.maximum(jnp.log(1.0 - p), -100.0)
  bce = -(t * logp + (1.0 - t) * log1p)
  w = jnp.where(t == 1.0, pw_ref[0, 0], 1.0)
  acc[1] += jnp.sum(w * bce)

  @pl.when(i == nblk - 1)
  def _():
    nn = float(N) * float(N)
    s = acc[0]
    norm_c = nn / ((nn - s) * 2.0)
    rec_ref[...] = (norm_c * (acc[1] / nn)).reshape(1, 1)
    kl_ref[...] = (acc[2] / float(N)).reshape(1, 1)
    zy = jnp.dot(oh_ref[...], z_s[...], precision=HIGH)
    oy_ref[...] = jnp.dot(zy, fw_ref[...], precision=HIGH) + fb_ref[...]


def _final(mu, lv, eps, c, onehot, fw, fb, pw):
  return pl.pallas_call(
      _final_kernel,
      grid=(N // _RB,),
      in_specs=[
          pl.BlockSpec((N, DP), lambda i: (0, 0)),
          pl.BlockSpec((N, DP), lambda i: (0, 0)),
          pl.BlockSpec((N, DP), lambda i: (0, 0)),
          pl.BlockSpec((_RB, N), lambda i: (i, 0)),
          pl.BlockSpec((B, N), lambda i: (0, 0)),
          pl.BlockSpec((DP, DP), lambda i: (0, 0)),
          pl.BlockSpec((1, DP), lambda i: (0, 0)),
          pl.BlockSpec((1, 1), lambda i: (0, 0)),
      ],
      out_specs=[
          pl.BlockSpec((B, DP), lambda i: (0, 0)),
          pl.BlockSpec((1, 1), lambda i: (0, 0)),
          pl.BlockSpec((1, 1), lambda i: (0, 0)),
      ],
      out_shape=[
          jax.ShapeDtypeStruct((B, DP), jnp.float32),
          jax.ShapeDtypeStruct((1, 1), jnp.float32),
          jax.ShapeDtypeStruct((1, 1), jnp.float32),
      ],
      scratch_shapes=[
          pltpu.VMEM((N, DP), jnp.float32),
          pltpu.SMEM((4,), jnp.float32),
      ],
  )(mu, lv, eps, c, onehot, fw, fb, pw)


def _pad2(a, r, c):
  return jnp.pad(a, ((0, r - a.shape[0]), (0, c - a.shape[1])))


def _gate_mats(wih, whh):
  h = wih.shape[0] // 3
  mats = []
  for w in (wih, whh):
    for g in range(3):
      mats.append(_pad2(w[g * h:(g + 1) * h].T.astype(jnp.float32), DP, DP))
  return mats


def _gate_biases(bih, bhh):
  h = bih.shape[0] // 3
  br = _pad2((bih[:h] + bhh[:h]).reshape(1, -1), 1, DP)
  bz = _pad2((bih[h:2 * h] + bhh[h:2 * h]).reshape(1, -1), 1, DP)
  bi = _pad2(bih[2 * h:].reshape(1, -1), 1, DP)
  bh = _pad2(bhh[2 * h:].reshape(1, -1), 1, DP)
  return [br, bz, bi, bh]


def kernel(user_feats, graph_node_features, graph_edge_index,
           merged_tree_feature, merged_tree_edge_index, indices,
           tweet_embedding, ue_w1, ue_b1, ue_w2, ue_b2, wih0, whh0, bih0,
           bhh0, wih1, whh1, bih1, bhh1, conv1_w, conv1_b, convm_w, convm_b,
           convv_w, convv_b, fc1_w, fc1_b):
  f32 = jnp.float32
  table = _pad2(tweet_embedding.astype(f32), VOCAB, DP)
  ids2d = graph_node_features.astype(jnp.int32).reshape(_NW, _CHUNKS, 128)
  emb = _sc_gather(table, ids2d)

  edges_flat = graph_edge_index.astype(jnp.int32).reshape(-1)
  cflat = _sc_build_c(edges_flat)
  c = cflat.reshape(N, N)

  h0 = jax.random.normal(jax.random.key(42), (2, NT, 100), f32)
  h00 = _pad2(h0[0], NT, DP)
  h01 = _pad2(h0[1], NT, DP)
  wm = _gate_mats(wih0, whh0) + _gate_mats(wih1, whh1)
  bv = _gate_biases(bih0, bhh0) + _gate_biases(bih1, bhh1)
  hn = _gru(emb, h00, h01, wm, bv)

  uf = _pad2(user_feats.astype(f32), NU, 16)
  user_emb = _user_mlp(
      uf,
      _pad2(ue_w1.T.astype(f32), 16, DP),
      _pad2(ue_b1.reshape(1, -1), 1, DP),
      _pad2(ue_w2.T.astype(f32), DP, DP),
      _pad2(ue_b2.reshape(1, -1), 1, DP),
  )

  bs = indices.shape[0]
  x_input = jnp.concatenate([hn[:bs], user_emb, hn[bs:]], axis=0)

  dinv_row, suma = _deg(c)
  dinv_col = dinv_row.reshape(N, 1)
  nn = float(N) * float(N)
  pw = (nn - suma) / suma

  x1 = _gcn_elu(c, x_input, _pad2(conv1_w.astype(f32), DP, DP),
                _pad2(conv1_b.reshape(1, -1), 1, DP), dinv_col)
  mu = _gcn_relu(c, x1, _pad2(convm_w.astype(f32), DP, DP),
                 _pad2(convm_b.reshape(1, -1), 1, DP), dinv_col)
  logvar = _gcn_relu(c, x1, _pad2(convv_w.astype(f32), DP, DP),
                     _pad2(convv_b.reshape(1, -1), 1, DP), dinv_col)

  eps = _pad2(jax.random.normal(jax.random.key(7), (N, 100), f32), N, DP)
  onehot = (indices.astype(jnp.int32)[:, None] ==
            jnp.arange(N, dtype=jnp.int32)[None, :]).astype(f32)
  oy128, kl11, rec11 = _final(
      mu, logvar, eps, c, onehot,
      _pad2(fc1_w.T.astype(f32), DP, DP),
      _pad2(fc1_b.reshape(1, -1), 1, DP), pw)

  out_y = oy128[:, :4]
  kl = kl11.reshape(1)
  rec = rec11.reshape(())
  return (out_y, kl, rec)


# speedup vs baseline: 2.9399x; 1.5000x over previous
import functools
import jax
import jax.numpy as jnp
from jax import lax
from jax.experimental import pallas as pl
from jax.experimental.pallas import tpu as pltpu
from jax.experimental.pallas import tpu_sc as plsc

N = 4096
NT = 3072
NU = 1024
E = 65536
SEQ = 24
VOCAB = 30000
B = 64
DP = 128
HIGH = lax.Precision.DEFAULT


_NW = 32
_ROWS = NT * SEQ
_RPW = _ROWS // _NW
_CHUNKS = _RPW // 128


def _sc_gather(table, ids2d):
  mesh = plsc.VectorSubcoreMesh(core_axis_name="c", subcore_axis_name="s")

  @functools.partial(
      pl.kernel,
      mesh=mesh,
      out_type=jax.ShapeDtypeStruct((_ROWS, DP), jnp.float32),
      scratch_types=[
          pltpu.VMEM((_CHUNKS, 128), jnp.int32),
          pltpu.VMEM((128, DP), jnp.float32),
          pltpu.SemaphoreType.DMA,
      ],
  )
  def k(table_hbm, idx_hbm, out_hbm, idx_v, rows_v, sem):
    wid = lax.axis_index("s") * 2 + lax.axis_index("c")
    pltpu.sync_copy(idx_hbm.at[wid], idx_v)
    for c in range(_CHUNKS):
      pltpu.async_copy(table_hbm.at[idx_v.at[c]], rows_v, sem).wait()
      pltpu.sync_copy(
          rows_v, out_hbm.at[pl.ds(wid * _RPW + c * 128, 128)])

  return k(table, ids2d)


_BLK_ELEMS = 256 * N
_EPT = E // 16
_ZB = 16384


def _sc_build_c(edges_flat):
  mesh = plsc.VectorSubcoreMesh(core_axis_name="c", subcore_axis_name="s")

  @functools.partial(
      pl.kernel,
      mesh=mesh,
      out_type=jax.ShapeDtypeStruct((N * N,), jnp.float32),
      scratch_types=[
          pltpu.VMEM((_EPT,), jnp.int32),
          pltpu.VMEM((_EPT,), jnp.int32),
          pltpu.VMEM((_EPT,), jnp.int32),
          pltpu.VMEM((32, 128), jnp.int32),
          pltpu.VMEM((_EPT,), jnp.float32),
          pltpu.VMEM((_ZB,), jnp.float32),
          pltpu.VMEM_SHARED((_BLK_ELEMS + 16,), jnp.float32),
          pltpu.SemaphoreType.DMA,
      ],
  )
  def k(edges_hbm, out_hbm, srcv, dstv, keyv, idxv, onesv, zerov, acc, sem):
    cid = lax.axis_index("c")
    sid = lax.axis_index("s")
    ebase = sid * _EPT
    pltpu.sync_copy(edges_hbm.at[pl.ds(ebase, _EPT)], srcv)
    pltpu.sync_copy(edges_hbm.at[pl.ds(E + ebase, _EPT)], dstv)

    def init_body(i, _):
      sl = pl.ds(i * 16, 16)
      keyv[sl] = srcv[sl] * N + dstv[sl]
      onesv[sl] = jnp.full((16,), 1.0, jnp.float32)
      return 0

    lax.fori_loop(0, _EPT // 16, init_body, 0)

    def zero_body(i, _):
      zerov[pl.ds(i * 16, 16)] = jnp.zeros((16,), jnp.float32)
      return 0

    lax.fori_loop(0, _ZB // 16, zero_body, 0)

    for p in range(8):
      blk = p * 2 + cid
      base = blk * _BLK_ELEMS
      zds = [
          pltpu.async_copy(
              zerov,
              acc.at[pl.ds(sid * (_BLK_ELEMS // 16) + q * _ZB, _ZB)], sem)
          for q in range(_BLK_ELEMS // 16 // _ZB)
      ]

      @pl.when(sid == 0)
      def _():
        pltpu.sync_copy(zerov.at[pl.ds(0, 16)], acc.at[pl.ds(_BLK_ELEMS, 16)])

      def mask_body(r, _):
        for q in range(8):
          sl = pl.ds(r * 128 + q * 16, 16)
          kk = keyv[sl] - base
          ok = (kk >= 0) & (kk < _BLK_ELEMS)
          idxv[r, pl.ds(q * 16, 16)] = jnp.where(
              ok, kk, jnp.full((16,), _BLK_ELEMS, jnp.int32))
        return 0

      lax.fori_loop(0, 32, mask_body, 0)
      for d in zds:
        d.wait()
      plsc.subcore_barrier()

      sds = [
          pltpu.async_copy(
              onesv.at[pl.ds(j * 128, 128)], acc.at[idxv.at[j]], sem,
              add=True)
          for j in range(32)
      ]
      for d in sds:
        d.wait()
      plsc.subcore_barrier()
      pltpu.sync_copy(
          acc.at[pl.ds(sid * (_BLK_ELEMS // 16), _BLK_ELEMS // 16)],
          out_hbm.at[pl.ds(base + sid * (_BLK_ELEMS // 16),
                           _BLK_ELEMS // 16)])
      plsc.subcore_barrier()

  return k(edges_flat)


def _user_mlp_kernel(uf_ref, w1_ref, b1_ref, w2_ref, b2_ref, out_ref):
  u = jnp.maximum(
      jnp.dot(uf_ref[...], w1_ref[...], precision=HIGH) + b1_ref[...], 0.0)
  out_ref[...] = jnp.dot(u, w2_ref[...], precision=HIGH) + b2_ref[...]


def _user_mlp(uf, w1t, b1, w2t, b2):
  return pl.pallas_call(
      _user_mlp_kernel,
      out_shape=jax.ShapeDtypeStruct((NU, DP), jnp.float32),
  )(uf, w1t, b1, w2t, b2)


_GB = 512


def _gru_kernel(emb_ref, h00_ref, h01_ref,
                w0r, w0z, w0n, u0r, u0z, u0n, b0r, b0z, b0i, b0h,
                w1r, w1z, w1n, u1r, u1z, u1n, b1r, b1z, b1i, b1h,
                hn_ref, out0_s):
  x = emb_ref[...].reshape(_GB, SEQ, DP)
  h = h00_ref[...]
  for t in range(SEQ):
    xt = x[:, t, :]
    r = jax.nn.sigmoid(jnp.dot(xt, w0r[...], precision=HIGH) +
                       jnp.dot(h, u0r[...], precision=HIGH) + b0r[...])
    z = jax.nn.sigmoid(jnp.dot(xt, w0z[...], precision=HIGH) +
                       jnp.dot(h, u0z[...], precision=HIGH) + b0z[...])
    n = jnp.tanh(jnp.dot(xt, w0n[...], precision=HIGH) + b0i[...] +
                 r * (jnp.dot(h, u0n[...], precision=HIGH) + b0h[...]))
    h = (1.0 - z) * n + z * h
    out0_s[:, t, :] = h
  h = h01_ref[...]
  for t in range(SEQ):
    ot = out0_s[:, t, :]
    r = jax.nn.sigmoid(jnp.dot(ot, w1r[...], precision=HIGH) +
                       jnp.dot(h, u1r[...], precision=HIGH) + b1r[...])
    z = jax.nn.sigmoid(jnp.dot(ot, w1z[...], precision=HIGH) +
                       jnp.dot(h, u1z[...], precision=HIGH) + b1z[...])
    n = jnp.tanh(jnp.dot(ot, w1n[...], precision=HIGH) + b1i[...] +
                 r * (jnp.dot(h, u1n[...], precision=HIGH) + b1h[...]))
    h = (1.0 - z) * n + z * h
  hn_ref[...] = h


def _gru(emb, h00, h01, wmats, bvecs):
  wspec = pl.BlockSpec((DP, DP), lambda j: (0, 0))
  bspec = pl.BlockSpec((1, DP), lambda j: (0, 0))
  return pl.pallas_call(
      _gru_kernel,
      grid=(NT // _GB,),
      in_specs=[
          pl.BlockSpec((_GB * SEQ, DP), lambda j: (j, 0)),
          pl.BlockSpec((_GB, DP), lambda j: (j, 0)),
          pl.BlockSpec((_GB, DP), lambda j: (j, 0)),
      ] + [wspec] * 6 + [bspec] * 4 + [wspec] * 6 + [bspec] * 4,
      out_specs=pl.BlockSpec((_GB, DP), lambda j: (j, 0)),
      out_shape=jax.ShapeDtypeStruct((NT, DP), jnp.float32),
      scratch_shapes=[pltpu.VMEM((_GB, SEQ, DP), jnp.float32)],
  )(emb, h00, h01, *wmats[:6], *bvecs[:4], *wmats[6:], *bvecs[4:])


_RB = 512


def _deg_kernel(c_ref, dinv_ref, suma_ref):
  j = pl.program_id(0)

  @pl.when(j == 0)
  def _():
    dinv_ref[...] = jnp.zeros_like(dinv_ref)

  dinv_ref[...] += jnp.sum(c_ref[...], axis=0, keepdims=True)

  @pl.when(j == N // _RB - 1)
  def _():
    cs = dinv_ref[...]
    suma_ref[...] = (jnp.sum(cs) + float(N)).reshape(1, 1)
    dinv_ref[...] = lax.rsqrt(cs + 1.0)


def _deg(c):
  return pl.pallas_call(
      _deg_kernel,
      grid=(N // _RB,),
      in_specs=[pl.BlockSpec((_RB, N), lambda j: (j, 0))],
      out_specs=[
          pl.BlockSpec((1, N), lambda j: (0, 0)),
          pl.BlockSpec((1, 1), lambda j: (0, 0)),
      ],
      out_shape=[
          jax.ShapeDtypeStruct((1, N), jnp.float32),
          jax.ShapeDtypeStruct((1, 1), jnp.float32),
      ],
  )(c)


def _make_gcn(act):
  def body(c_ref, x_ref, w_ref, b_ref, dall_ref, dblk_ref, out_ref, hd_s):
    j = pl.program_id(0)

    @pl.when(j == 0)
    def _():
      hd_s[...] = jnp.dot(x_ref[...], w_ref[...],
                          precision=HIGH) * dall_ref[...]

    y = lax.dot_general(c_ref[...], hd_s[...], (((0,), (0,)), ((), ())),
                        precision=HIGH)
    y = (y + hd_s[pl.ds(j * _RB, _RB), :]) * dblk_ref[...] + b_ref[...]
    out_ref[...] = act(y)

  def run(c, x, w, b, dinv_col):
    return pl.pallas_call(
        body,
        grid=(N // _RB,),
        in_specs=[
            pl.BlockSpec((N, _RB), lambda j: (0, j)),
            pl.BlockSpec((N, DP), lambda j: (0, 0)),
            pl.BlockSpec((DP, DP), lambda j: (0, 0)),
            pl.BlockSpec((1, DP), lambda j: (0, 0)),
            pl.BlockSpec((N, 1), lambda j: (0, 0)),
            pl.BlockSpec((_RB, 1), lambda j: (j, 0)),
        ],
        out_specs=pl.BlockSpec((_RB, DP), lambda j: (j, 0)),
        out_shape=jax.ShapeDtypeStruct((N, DP), jnp.float32),
        scratch_shapes=[pltpu.VMEM((N, DP), jnp.float32)],
    )(c, x, w, b, dinv_col, dinv_col)

  return run


_gcn_elu = _make_gcn(lambda y: jnp.where(y > 0, y, jnp.exp(y) - 1.0))
_gcn_relu = _make_gcn(lambda y: jnp.maximum(y, 0.0))


def _final_kernel(mu_ref, lv_ref, eps_ref, c_ref, oh_ref, fw_ref, fb_ref,
                  pw_ref, oy_ref, kl_ref, rec_ref, z_s, acc):
  i = pl.program_id(0)
  nblk = N // _RB

  @pl.when(i == 0)
  def _():
    mu = mu_ref[...]
    lv = lv_ref[...]
    z_s[...] = mu + jnp.exp(lv * 0.5) * eps_ref[...]
    klds = -0.5 * (1.0 + lv - mu * mu - jnp.exp(lv))
    acc[0] = 0.0
    acc[1] = 0.0
    acc[2] = jnp.sum(klds)

  zb = z_s[pl.ds(i * _RB, _RB), :]
  logits = lax.dot_general(zb, z_s[...], (((1,), (1,)), ((), ())),
                           precision=HIGH)
  p = jax.nn.sigmoid(logits)
  acc[0] += jnp.sum(p)
  cols = lax.broadcasted_iota(jnp.int32, (_RB, N), 1)
  rows = lax.broadcasted_iota(jnp.int32, (_RB, N), 0) + i * _RB
  t = c_ref[...] + jnp.where(cols == rows, 1.0, 0.0)
  logp = jnp.maximum(jnp.log(p), -100.0)
  log1p = jnp.maximum(jnp.log(1.0 - p), -100.0)
  bce = -(t * logp + (1.0 - t) * log1p)
  w = jnp.where(t == 1.0, pw_ref[0, 0], 1.0)
  acc[1] += jnp.sum(w * bce)

  @pl.when(i == nblk - 1)
  def _():
    nn = float(N) * float(N)
    s = acc[0]
    norm_c = nn / ((nn - s) * 2.0)
    rec_ref[...] = (norm_c * (acc[1] / nn)).reshape(1, 1)
    kl_ref[...] = (acc[2] / float(N)).reshape(1, 1)
    zy = jnp.dot(oh_ref[...], z_s[...], precision=HIGH)
    oy_ref[...] = jnp.dot(zy, fw_ref[...], precision=HIGH) + fb_ref[...]


def _final(mu, lv, eps, c, onehot, fw, fb, pw):
  return pl.pallas_call(
      _final_kernel,
      grid=(N // _RB,),
      in_specs=[
          pl.BlockSpec((N, DP), lambda i: (0, 0)),
          pl.BlockSpec((N, DP), lambda i: (0, 0)),
          pl.BlockSpec((N, DP), lambda i: (0, 0)),
          pl.BlockSpec((_RB, N), lambda i: (i, 0)),
          pl.BlockSpec((B, N), lambda i: (0, 0)),
          pl.BlockSpec((DP, DP), lambda i: (0, 0)),
          pl.BlockSpec((1, DP), lambda i: (0, 0)),
          pl.BlockSpec((1, 1), lambda i: (0, 0)),
      ],
      out_specs=[
          pl.BlockSpec((B, DP), lambda i: (0, 0)),
          pl.BlockSpec((1, 1), lambda i: (0, 0)),
          pl.BlockSpec((1, 1), lambda i: (0, 0)),
      ],
      out_shape=[
          jax.ShapeDtypeStruct((B, DP), jnp.float32),
          jax.ShapeDtypeStruct((1, 1), jnp.float32),
          jax.ShapeDtypeStruct((1, 1), jnp.float32),
      ],
      scratch_shapes=[
          pltpu.VMEM((N, DP), jnp.float32),
          pltpu.SMEM((4,), jnp.float32),
      ],
  )(mu, lv, eps, c, onehot, fw, fb, pw)


def _pad2(a, r, c):
  return jnp.pad(a, ((0, r - a.shape[0]), (0, c - a.shape[1])))


def _gate_mats(wih, whh):
  h = wih.shape[0] // 3
  mats = []
  for w in (wih, whh):
    for g in range(3):
      mats.append(_pad2(w[g * h:(g + 1) * h].T.astype(jnp.float32), DP, DP))
  return mats


def _gate_biases(bih, bhh):
  h = bih.shape[0] // 3
  br = _pad2((bih[:h] + bhh[:h]).reshape(1, -1), 1, DP)
  bz = _pad2((bih[h:2 * h] + bhh[h:2 * h]).reshape(1, -1), 1, DP)
  bi = _pad2(bih[2 * h:].reshape(1, -1), 1, DP)
  bh = _pad2(bhh[2 * h:].reshape(1, -1), 1, DP)
  return [br, bz, bi, bh]


def kernel(user_feats, graph_node_features, graph_edge_index,
           merged_tree_feature, merged_tree_edge_index, indices,
           tweet_embedding, ue_w1, ue_b1, ue_w2, ue_b2, wih0, whh0, bih0,
           bhh0, wih1, whh1, bih1, bhh1, conv1_w, conv1_b, convm_w, convm_b,
           convv_w, convv_b, fc1_w, fc1_b):
  f32 = jnp.float32
  table = _pad2(tweet_embedding.astype(f32), VOCAB, DP)
  ids2d = graph_node_features.astype(jnp.int32).reshape(_NW, _CHUNKS, 128)
  emb = _sc_gather(table, ids2d)

  edges_flat = graph_edge_index.astype(jnp.int32).reshape(-1)
  cflat = _sc_build_c(edges_flat)
  c = cflat.reshape(N, N)

  h0 = jax.random.normal(jax.random.key(42), (2, NT, 100), f32)
  h00 = _pad2(h0[0], NT, DP)
  h01 = _pad2(h0[1], NT, DP)
  wm = _gate_mats(wih0, whh0) + _gate_mats(wih1, whh1)
  bv = _gate_biases(bih0, bhh0) + _gate_biases(bih1, bhh1)
  hn = _gru(emb, h00, h01, wm, bv)

  uf = _pad2(user_feats.astype(f32), NU, 16)
  user_emb = _user_mlp(
      uf,
      _pad2(ue_w1.T.astype(f32), 16, DP),
      _pad2(ue_b1.reshape(1, -1), 1, DP),
      _pad2(ue_w2.T.astype(f32), DP, DP),
      _pad2(ue_b2.reshape(1, -1), 1, DP),
  )

  bs = indices.shape[0]
  x_input = jnp.concatenate([hn[:bs], user_emb, hn[bs:]], axis=0)

  dinv_row, suma = _deg(c)
  dinv_col = dinv_row.reshape(N, 1)
  nn = float(N) * float(N)
  pw = (nn - suma) / suma

  x1 = _gcn_elu(c, x_input, _pad2(conv1_w.astype(f32), DP, DP),
                _pad2(conv1_b.reshape(1, -1), 1, DP), dinv_col)
  mu = _gcn_relu(c, x1, _pad2(convm_w.astype(f32), DP, DP),
                 _pad2(convm_b.reshape(1, -1), 1, DP), dinv_col)
  logvar = _gcn_relu(c, x1, _pad2(convv_w.astype(f32), DP, DP),
                     _pad2(convv_b.reshape(1, -1), 1, DP), dinv_col)

  eps = _pad2(jax.random.normal(jax.random.key(7), (N, 100), f32), N, DP)
  onehot = (indices.astype(jnp.int32)[:, None] ==
            jnp.arange(N, dtype=jnp.int32)[None, :]).astype(f32)
  oy128, kl11, rec11 = _final(
      mu, logvar, eps, c, onehot,
      _pad2(fc1_w.T.astype(f32), DP, DP),
      _pad2(fc1_b.reshape(1, -1), 1, DP), pw)

  out_y = oy128[:, :4]
  kl = kl11.reshape(1)
  rec = rec11.reshape(())
  return (out_y, kl, rec)


# speedup vs baseline: 2.9429x; 1.0010x over previous
import functools
import jax
import jax.numpy as jnp
from jax import lax
from jax.experimental import pallas as pl
from jax.experimental.pallas import tpu as pltpu
from jax.experimental.pallas import tpu_sc as plsc

N = 4096
NT = 3072
NU = 1024
E = 65536
SEQ = 24
VOCAB = 30000
B = 64
DP = 128
HIGH = lax.Precision.DEFAULT


_NW = 32
_ROWS = NT * SEQ
_RPW = _ROWS // _NW
_CHUNKS = _RPW // 128


def _sc_gather(table, ids2d):
  mesh = plsc.VectorSubcoreMesh(core_axis_name="c", subcore_axis_name="s")

  @functools.partial(
      pl.kernel,
      mesh=mesh,
      out_type=jax.ShapeDtypeStruct((_ROWS, DP), jnp.float32),
      scratch_types=[
          pltpu.VMEM((_RPW,), jnp.int32),
          pltpu.VMEM((768, DP), jnp.float32),
          pltpu.SemaphoreType.DMA,
      ],
  )
  def k(table_hbm, idx_hbm, out_hbm, idx_v, rows_v, sem):
    wid = lax.axis_index("s") * 2 + lax.axis_index("c")
    pltpu.sync_copy(idx_hbm.at[pl.ds(wid * _RPW, _RPW)], idx_v)
    for c in range(3):
      pltpu.async_copy(
          table_hbm.at[idx_v.at[pl.ds(c * 768, 768)]], rows_v, sem).wait()
      pltpu.sync_copy(
          rows_v, out_hbm.at[pl.ds(wid * _RPW + c * 768, 768)])

  return k(table, ids2d)


_BLK_ELEMS = 256 * N
_EPT = E // 16
_ZB = 16384


def _sc_build_c(edges_flat):
  mesh = plsc.VectorSubcoreMesh(core_axis_name="c", subcore_axis_name="s")

  @functools.partial(
      pl.kernel,
      mesh=mesh,
      out_type=jax.ShapeDtypeStruct((N * N,), jnp.float32),
      scratch_types=[
          pltpu.VMEM((_EPT,), jnp.int32),
          pltpu.VMEM((_EPT,), jnp.int32),
          pltpu.VMEM((_EPT,), jnp.int32),
          pltpu.VMEM((_EPT,), jnp.int32),
          pltpu.VMEM((_EPT,), jnp.float32),
          pltpu.VMEM((_ZB,), jnp.float32),
          pltpu.VMEM_SHARED((_BLK_ELEMS + 16,), jnp.float32),
          pltpu.SemaphoreType.DMA,
      ],
  )
  def k(edges_hbm, out_hbm, srcv, dstv, keyv, idxv, onesv, zerov, acc, sem):
    cid = lax.axis_index("c")
    sid = lax.axis_index("s")
    ebase = sid * _EPT
    pltpu.sync_copy(edges_hbm.at[pl.ds(ebase, _EPT)], srcv)
    pltpu.sync_copy(edges_hbm.at[pl.ds(E + ebase, _EPT)], dstv)

    def init_body(i, _):
      sl = pl.ds(i * 16, 16)
      keyv[sl] = srcv[sl] * N + dstv[sl]
      onesv[sl] = jnp.full((16,), 1.0, jnp.float32)
      return 0

    lax.fori_loop(0, _EPT // 16, init_body, 0)

    def zero_body(i, _):
      zerov[pl.ds(i * 16, 16)] = jnp.zeros((16,), jnp.float32)
      return 0

    lax.fori_loop(0, _ZB // 16, zero_body, 0)

    for p in range(8):
      blk = p * 2 + cid
      base = blk * _BLK_ELEMS
      zds = [
          pltpu.async_copy(
              zerov,
              acc.at[pl.ds(sid * (_BLK_ELEMS // 16) + q * _ZB, _ZB)], sem)
          for q in range(_BLK_ELEMS // 16 // _ZB)
      ]

      @pl.when(sid == 0)
      def _():
        pltpu.sync_copy(zerov.at[pl.ds(0, 16)], acc.at[pl.ds(_BLK_ELEMS, 16)])

      def mask_body(i, _):
        sl = pl.ds(i * 16, 16)
        kk = keyv[sl] - base
        ok = (kk >= 0) & (kk < _BLK_ELEMS)
        idxv[sl] = jnp.where(
            ok, kk, jnp.full((16,), _BLK_ELEMS, jnp.int32))
        return 0

      lax.fori_loop(0, _EPT // 16, mask_body, 0)
      for d in zds:
        d.wait()
      plsc.subcore_barrier()

      pltpu.sync_copy(onesv, acc.at[idxv], add=True)
      plsc.subcore_barrier()
      pltpu.sync_copy(
          acc.at[pl.ds(sid * (_BLK_ELEMS // 16), _BLK_ELEMS // 16)],
          out_hbm.at[pl.ds(base + sid * (_BLK_ELEMS // 16),
                           _BLK_ELEMS // 16)])
      plsc.subcore_barrier()

  return k(edges_flat)


def _user_mlp_kernel(uf_ref, w1_ref, b1_ref, w2_ref, b2_ref, out_ref):
  u = jnp.maximum(
      jnp.dot(uf_ref[...], w1_ref[...], precision=HIGH) + b1_ref[...], 0.0)
  out_ref[...] = jnp.dot(u, w2_ref[...], precision=HIGH) + b2_ref[...]


def _user_mlp(uf, w1t, b1, w2t, b2):
  return pl.pallas_call(
      _user_mlp_kernel,
      out_shape=jax.ShapeDtypeStruct((NU, DP), jnp.float32),
  )(uf, w1t, b1, w2t, b2)


_GB = 512


def _gru_kernel(emb_ref, h00_ref, h01_ref,
                w0r, w0z, w0n, u0r, u0z, u0n, b0r, b0z, b0i, b0h,
                w1r, w1z, w1n, u1r, u1z, u1n, b1r, b1z, b1i, b1h,
                hn_ref, out0_s):
  x = emb_ref[...].reshape(_GB, SEQ, DP)
  h = h00_ref[...]
  for t in range(SEQ):
    xt = x[:, t, :]
    r = jax.nn.sigmoid(jnp.dot(xt, w0r[...], precision=HIGH) +
                       jnp.dot(h, u0r[...], precision=HIGH) + b0r[...])
    z = jax.nn.sigmoid(jnp.dot(xt, w0z[...], precision=HIGH) +
                       jnp.dot(h, u0z[...], precision=HIGH) + b0z[...])
    n = jnp.tanh(jnp.dot(xt, w0n[...], precision=HIGH) + b0i[...] +
                 r * (jnp.dot(h, u0n[...], precision=HIGH) + b0h[...]))
    h = (1.0 - z) * n + z * h
    out0_s[:, t, :] = h
  h = h01_ref[...]
  for t in range(SEQ):
    ot = out0_s[:, t, :]
    r = jax.nn.sigmoid(jnp.dot(ot, w1r[...], precision=HIGH) +
                       jnp.dot(h, u1r[...], precision=HIGH) + b1r[...])
    z = jax.nn.sigmoid(jnp.dot(ot, w1z[...], precision=HIGH) +
                       jnp.dot(h, u1z[...], precision=HIGH) + b1z[...])
    n = jnp.tanh(jnp.dot(ot, w1n[...], precision=HIGH) + b1i[...] +
                 r * (jnp.dot(h, u1n[...], precision=HIGH) + b1h[...]))
    h = (1.0 - z) * n + z * h
  hn_ref[...] = h


def _gru(emb, h00, h01, wmats, bvecs):
  wspec = pl.BlockSpec((DP, DP), lambda j: (0, 0))
  bspec = pl.BlockSpec((1, DP), lambda j: (0, 0))
  return pl.pallas_call(
      _gru_kernel,
      grid=(NT // _GB,),
      in_specs=[
          pl.BlockSpec((_GB * SEQ, DP), lambda j: (j, 0)),
          pl.BlockSpec((_GB, DP), lambda j: (j, 0)),
          pl.BlockSpec((_GB, DP), lambda j: (j, 0)),
      ] + [wspec] * 6 + [bspec] * 4 + [wspec] * 6 + [bspec] * 4,
      out_specs=pl.BlockSpec((_GB, DP), lambda j: (j, 0)),
      out_shape=jax.ShapeDtypeStruct((NT, DP), jnp.float32),
      scratch_shapes=[pltpu.VMEM((_GB, SEQ, DP), jnp.float32)],
  )(emb, h00, h01, *wmats[:6], *bvecs[:4], *wmats[6:], *bvecs[4:])


_RB = 512


def _deg_kernel(c_ref, dinv_ref, suma_ref):
  j = pl.program_id(0)

  @pl.when(j == 0)
  def _():
    dinv_ref[...] = jnp.zeros_like(dinv_ref)

  dinv_ref[...] += jnp.sum(c_ref[...], axis=0, keepdims=True)

  @pl.when(j == N // _RB - 1)
  def _():
    cs = dinv_ref[...]
    suma_ref[...] = (jnp.sum(cs) + float(N)).reshape(1, 1)
    dinv_ref[...] = lax.rsqrt(cs + 1.0)


def _deg(c):
  return pl.pallas_call(
      _deg_kernel,
      grid=(N // _RB,),
      in_specs=[pl.BlockSpec((_RB, N), lambda j: (j, 0))],
      out_specs=[
          pl.BlockSpec((1, N), lambda j: (0, 0)),
          pl.BlockSpec((1, 1), lambda j: (0, 0)),
      ],
      out_shape=[
          jax.ShapeDtypeStruct((1, N), jnp.float32),
          jax.ShapeDtypeStruct((1, 1), jnp.float32),
      ],
  )(c)


def _make_gcn(act):
  def body(c_ref, x_ref, w_ref, b_ref, dall_ref, dblk_ref, out_ref, hd_s):
    j = pl.program_id(0)

    @pl.when(j == 0)
    def _():
      hd_s[...] = jnp.dot(x_ref[...], w_ref[...],
                          precision=HIGH) * dall_ref[...]

    y = lax.dot_general(c_ref[...], hd_s[...], (((0,), (0,)), ((), ())),
                        precision=HIGH)
    y = (y + hd_s[pl.ds(j * _RB, _RB), :]) * dblk_ref[...] + b_ref[...]
    out_ref[...] = act(y)

  def run(c, x, w, b, dinv_col):
    return pl.pallas_call(
        body,
        grid=(N // _RB,),
        in_specs=[
            pl.BlockSpec((N, _RB), lambda j: (0, j)),
            pl.BlockSpec((N, DP), lambda j: (0, 0)),
            pl.BlockSpec((DP, DP), lambda j: (0, 0)),
            pl.BlockSpec((1, DP), lambda j: (0, 0)),
            pl.BlockSpec((N, 1), lambda j: (0, 0)),
            pl.BlockSpec((_RB, 1), lambda j: (j, 0)),
        ],
        out_specs=pl.BlockSpec((_RB, DP), lambda j: (j, 0)),
        out_shape=jax.ShapeDtypeStruct((N, DP), jnp.float32),
        scratch_shapes=[pltpu.VMEM((N, DP), jnp.float32)],
    )(c, x, w, b, dinv_col, dinv_col)

  return run


_gcn_elu = _make_gcn(lambda y: jnp.where(y > 0, y, jnp.exp(y) - 1.0))
_gcn_relu = _make_gcn(lambda y: jnp.maximum(y, 0.0))


def _final_kernel(mu_ref, lv_ref, eps_ref, c_ref, oh_ref, fw_ref, fb_ref,
                  pw_ref, oy_ref, kl_ref, rec_ref, z_s, acc):
  i = pl.program_id(0)
  nblk = N // _RB

  @pl.when(i == 0)
  def _():
    mu = mu_ref[...]
    lv = lv_ref[...]
    z_s[...] = mu + jnp.exp(lv * 0.5) * eps_ref[...]
    klds = -0.5 * (1.0 + lv - mu * mu - jnp.exp(lv))
    acc[0] = 0.0
    acc[1] = 0.0
    acc[2] = jnp.sum(klds)

  zb = z_s[pl.ds(i * _RB, _RB), :]
  logits = lax.dot_general(zb, z_s[...], (((1,), (1,)), ((), ())),
                           precision=HIGH)
  p = jax.nn.sigmoid(logits)
  acc[0] += jnp.sum(p)
  cols = lax.broadcasted_iota(jnp.int32, (_RB, N), 1)
  rows = lax.broadcasted_iota(jnp.int32, (_RB, N), 0) + i * _RB
  t = c_ref[...] + jnp.where(cols == rows, 1.0, 0.0)
  logp = jnp.maximum(jnp.log(p), -100.0)
  log1p = jnp.maximum(jnp.log(1.0 - p), -100.0)
  bce = -(t * logp + (1.0 - t) * log1p)
  w = jnp.where(t == 1.0, pw_ref[0, 0], 1.0)
  acc[1] += jnp.sum(w * bce)

  @pl.when(i == nblk - 1)
  def _():
    nn = float(N) * float(N)
    s = acc[0]
    norm_c = nn / ((nn - s) * 2.0)
    rec_ref[...] = (norm_c * (acc[1] / nn)).reshape(1, 1)
    kl_ref[...] = (acc[2] / float(N)).reshape(1, 1)
    zy = jnp.dot(oh_ref[...], z_s[...], precision=HIGH)
    oy_ref[...] = jnp.dot(zy, fw_ref[...], precision=HIGH) + fb_ref[...]


def _final(mu, lv, eps, c, onehot, fw, fb, pw):
  return pl.pallas_call(
      _final_kernel,
      grid=(N // _RB,),
      in_specs=[
          pl.BlockSpec((N, DP), lambda i: (0, 0)),
          pl.BlockSpec((N, DP), lambda i: (0, 0)),
          pl.BlockSpec((N, DP), lambda i: (0, 0)),
          pl.BlockSpec((_RB, N), lambda i: (i, 0)),
          pl.BlockSpec((B, N), lambda i: (0, 0)),
          pl.BlockSpec((DP, DP), lambda i: (0, 0)),
          pl.BlockSpec((1, DP), lambda i: (0, 0)),
          pl.BlockSpec((1, 1), lambda i: (0, 0)),
      ],
      out_specs=[
          pl.BlockSpec((B, DP), lambda i: (0, 0)),
          pl.BlockSpec((1, 1), lambda i: (0, 0)),
          pl.BlockSpec((1, 1), lambda i: (0, 0)),
      ],
      out_shape=[
          jax.ShapeDtypeStruct((B, DP), jnp.float32),
          jax.ShapeDtypeStruct((1, 1), jnp.float32),
          jax.ShapeDtypeStruct((1, 1), jnp.float32),
      ],
      scratch_shapes=[
          pltpu.VMEM((N, DP), jnp.float32),
          pltpu.SMEM((4,), jnp.float32),
      ],
  )(mu, lv, eps, c, onehot, fw, fb, pw)


def _pad2(a, r, c):
  return jnp.pad(a, ((0, r - a.shape[0]), (0, c - a.shape[1])))


def _gate_mats(wih, whh):
  h = wih.shape[0] // 3
  mats = []
  for w in (wih, whh):
    for g in range(3):
      mats.append(_pad2(w[g * h:(g + 1) * h].T.astype(jnp.float32), DP, DP))
  return mats


def _gate_biases(bih, bhh):
  h = bih.shape[0] // 3
  br = _pad2((bih[:h] + bhh[:h]).reshape(1, -1), 1, DP)
  bz = _pad2((bih[h:2 * h] + bhh[h:2 * h]).reshape(1, -1), 1, DP)
  bi = _pad2(bih[2 * h:].reshape(1, -1), 1, DP)
  bh = _pad2(bhh[2 * h:].reshape(1, -1), 1, DP)
  return [br, bz, bi, bh]


def kernel(user_feats, graph_node_features, graph_edge_index,
           merged_tree_feature, merged_tree_edge_index, indices,
           tweet_embedding, ue_w1, ue_b1, ue_w2, ue_b2, wih0, whh0, bih0,
           bhh0, wih1, whh1, bih1, bhh1, conv1_w, conv1_b, convm_w, convm_b,
           convv_w, convv_b, fc1_w, fc1_b):
  f32 = jnp.float32
  table = _pad2(tweet_embedding.astype(f32), VOCAB, DP)
  ids2d = graph_node_features.astype(jnp.int32).reshape(-1)
  emb = _sc_gather(table, ids2d)

  edges_flat = graph_edge_index.astype(jnp.int32).reshape(-1)
  cflat = _sc_build_c(edges_flat)
  c = cflat.reshape(N, N)

  h0 = jax.random.normal(jax.random.key(42), (2, NT, 100), f32)
  h00 = _pad2(h0[0], NT, DP)
  h01 = _pad2(h0[1], NT, DP)
  wm = _gate_mats(wih0, whh0) + _gate_mats(wih1, whh1)
  bv = _gate_biases(bih0, bhh0) + _gate_biases(bih1, bhh1)
  hn = _gru(emb, h00, h01, wm, bv)

  uf = _pad2(user_feats.astype(f32), NU, 16)
  user_emb = _user_mlp(
      uf,
      _pad2(ue_w1.T.astype(f32), 16, DP),
      _pad2(ue_b1.reshape(1, -1), 1, DP),
      _pad2(ue_w2.T.astype(f32), DP, DP),
      _pad2(ue_b2.reshape(1, -1), 1, DP),
  )

  bs = indices.shape[0]
  x_input = jnp.concatenate([hn[:bs], user_emb, hn[bs:]], axis=0)

  dinv_row, suma = _deg(c)
  dinv_col = dinv_row.reshape(N, 1)
  nn = float(N) * float(N)
  pw = (nn - suma) / suma

  x1 = _gcn_elu(c, x_input, _pad2(conv1_w.astype(f32), DP, DP),
                _pad2(conv1_b.reshape(1, -1), 1, DP), dinv_col)
  mu = _gcn_relu(c, x1, _pad2(convm_w.astype(f32), DP, DP),
                 _pad2(convm_b.reshape(1, -1), 1, DP), dinv_col)
  logvar = _gcn_relu(c, x1, _pad2(convv_w.astype(f32), DP, DP),
                     _pad2(convv_b.reshape(1, -1), 1, DP), dinv_col)

  eps = _pad2(jax.random.normal(jax.random.key(7), (N, 100), f32), N, DP)
  onehot = (indices.astype(jnp.int32)[:, None] ==
            jnp.arange(N, dtype=jnp.int32)[None, :]).astype(f32)
  oy128, kl11, rec11 = _final(
      mu, logvar, eps, c, onehot,
      _pad2(fc1_w.T.astype(f32), DP, DP),
      _pad2(fc1_b.reshape(1, -1), 1, DP), pw)

  out_y = oy128[:, :4]
  kl = kl11.reshape(1)
  rec = rec11.reshape(())
  return (out_y, kl, rec)
